# speedup vs baseline: 2.3137x; 2.3137x over previous
import jax
import jax.numpy as jnp
from jax.experimental import pallas as pl
from jax.experimental.pallas import tpu as pltpu

_B, _T, _C, _H, _E = 8, 4, 768, 2048, 8
_N = _B * _T
_HE = _H * _E
_BLK = 2048
_NBLK = _HE // _BLK


def _moe_body(x_ref, wg_ref, wfc_ref, wp_ref, o_ref, w_scr):
    j = pl.program_id(0)

    @pl.when(j == 0)
    def _():
        gate = jnp.dot(x_ref[...], wg_ref[...],
                       preferred_element_type=jnp.float32)
        e_iota = jax.lax.broadcasted_iota(jnp.int32, (_N, _E), 1)
        i1 = jnp.argmax(gate, axis=-1)
        is1 = e_iota == i1[:, None]
        m1 = jnp.max(gate, axis=-1, keepdims=True)
        gate2 = jnp.where(is1, -jnp.inf, gate)
        i2 = jnp.argmax(gate2, axis=-1)
        is2 = e_iota == i2[:, None]
        m2 = jnp.max(gate2, axis=-1, keepdims=True)
        w_scr[...] = jnp.where(is1, m1, 0.0) + jnp.where(is2, m2, 0.0)

    h = jnp.dot(x_ref[...], wfc_ref[...],
                preferred_element_type=jnp.float32)
    h = jax.nn.gelu(h, approximate=True)
    w = w_scr[...]
    col_e = jax.lax.broadcasted_iota(jnp.int32, (_N, _BLK), 1) % _E
    wm = jnp.zeros((_N, _BLK), jnp.float32)
    for e in range(_E):
        wm = wm + jnp.where(col_e == e, w[:, e][:, None], 0.0)
    h = h * wm

    part = jnp.dot(h, wp_ref[...], preferred_element_type=jnp.float32)

    @pl.when(j == 0)
    def _():
        o_ref[...] = part

    @pl.when(j > 0)
    def _():
        o_ref[...] = o_ref[...] + part


def _moe(x2, W_gate, W_fc, Wp, interpret=False):
    return pl.pallas_call(
        _moe_body,
        grid=(_NBLK,),
        in_specs=[
            pl.BlockSpec((_N, _C), lambda j: (0, 0)),
            pl.BlockSpec((_C, _E), lambda j: (0, 0)),
            pl.BlockSpec((_C, _BLK), lambda j: (0, j)),
            pl.BlockSpec((_BLK, _C), lambda j: (j, 0)),
        ],
        out_specs=pl.BlockSpec((_N, _C), lambda j: (0, 0)),
        out_shape=jax.ShapeDtypeStruct((_N, _C), jnp.float32),
        scratch_shapes=[pltpu.VMEM((_N, _E), jnp.float32)],
        compiler_params=pltpu.CompilerParams(
            dimension_semantics=("arbitrary",),
        ),
        interpret=interpret,
    )(x2, W_gate, W_fc, Wp)


def kernel(x, W_fc, W_proj, W_gate):
    Bx, Tx, Cx = x.shape
    x2 = x.reshape(Bx * Tx, Cx)
    Wp = W_proj.reshape(_H, _C, _E).transpose(0, 2, 1).reshape(_HE, _C)
    o = _moe(x2, W_gate, W_fc, Wp)
    return o.reshape(Bx, Tx, Cx)
